# baseline (speedup 1.0000x reference)
import functools

import jax
import jax.numpy as jnp
from jax import lax
from jax.experimental import pallas as pl
from jax.experimental.pallas import tpu as pltpu
from jax.experimental.pallas import tpu_sc as plsc

GM = 3.5
L = 16
CH = 128


@functools.partial(jax.jit, static_argnums=(0, 1, 2, 3))
def _biased_mf(B, K, NC, NS, user_ids, item_ids, bu, bv, U, V):
    NW = NC * NS
    bpw = B // NW
    nch = bpw // CH
    gpc = CH // L

    mesh = plsc.VectorSubcoreMesh(core_axis_name="c", subcore_axis_name="s")

    @functools.partial(
        pl.kernel,
        mesh=mesh,
        out_type=jax.ShapeDtypeStruct((B,), jnp.float32),
        compiler_params=pltpu.CompilerParams(needs_layout_passes=False,
                                             use_tc_tiling_on_sc=False),
        scratch_types=[
            pltpu.VMEM((nch, CH), jnp.int32),
            pltpu.VMEM((nch, CH), jnp.int32),
            pltpu.VMEM((bpw, K), jnp.float32),
            pltpu.VMEM((bpw, K), jnp.float32),
            pltpu.VMEM((bpw,), jnp.float32),
            pltpu.VMEM((bpw,), jnp.float32),
            pltpu.VMEM((bpw,), jnp.float32),
            pltpu.SemaphoreType.DMA,
        ],
    )
    def k(uids, iids, bu_t, bv_t, u_t, v_t, out,
          uidx, iidx, urow, vrow, bug, bvg, outv, sem):
        wid = lax.axis_index("s") * NC + lax.axis_index("c")
        base = wid * bpw

        for c in range(nch):
            pltpu.sync_copy(uids.at[pl.ds(base + c * CH, CH)], uidx.at[c])
            pltpu.sync_copy(iids.at[pl.ds(base + c * CH, CH)], iidx.at[c])

        copies = []
        for c in range(nch):
            copies.append(pltpu.async_copy(
                u_t.at[uidx.at[c]], urow.at[pl.ds(c * CH, CH), :], sem))
            copies.append(pltpu.async_copy(
                v_t.at[iidx.at[c]], vrow.at[pl.ds(c * CH, CH), :], sem))
            copies.append(pltpu.async_copy(
                bu_t.at[uidx.at[c]], bug.at[pl.ds(c * CH, CH)], sem))
            copies.append(pltpu.async_copy(
                bv_t.at[iidx.at[c]], bvg.at[pl.ds(c * CH, CH)], sem))
        for cp in copies:
            cp.wait()

        iota = lax.iota(jnp.int32, L)

        def group_body(g, _):
            rows = g * L + iota

            def body(j, acc):
                j16 = jnp.full((L,), j, jnp.int32)
                u16 = plsc.load_gather(urow, [rows, j16])
                v16 = plsc.load_gather(vrow, [rows, j16])
                return acc + u16 * v16

            acc = lax.fori_loop(0, K, body, jnp.zeros((L,), jnp.float32))
            off = g * L
            outv[pl.ds(off, L)] = (acc + bug[pl.ds(off, L)]
                                   + bvg[pl.ds(off, L)] + GM)
            return 0

        lax.fori_loop(0, bpw // L, group_body, 0)

        pltpu.sync_copy(outv, out.at[pl.ds(base, bpw)])

    return k(user_ids, item_ids, bu, bv, U, V)


def kernel(user_ids, item_ids, bu, bv, U, V):
    B = user_ids.shape[0]
    K = U.shape[1]
    info = plsc.get_sparse_core_info()
    return _biased_mf(B, K, info.num_cores, info.num_subcores,
                      user_ids.astype(jnp.int32), item_ids.astype(jnp.int32),
                      bu.reshape(-1), bv.reshape(-1), U, V)


# speedup vs baseline: 2.8223x; 2.8223x over previous
import functools

import jax
import jax.numpy as jnp
from jax import lax
from jax.experimental import pallas as pl
from jax.experimental.pallas import tpu as pltpu
from jax.experimental.pallas import tpu_sc as plsc

GM = 3.5
L = 16
CH = 128
RES = 8
W = 131072


def _detile(t):
    K, N = t.shape
    ng = K // RES
    nw = -(-N // W)

    def body(in_ref, *out_refs):
        for r in range(RES):
            out_refs[r][...] = in_ref[r, :]

    return pl.pallas_call(
        body,
        grid=(ng, nw),
        in_specs=[pl.BlockSpec((RES, W), lambda g, w: (g, w))],
        out_specs=[pl.BlockSpec((W,), lambda g, w: (g * nw + w))
                   for _ in range(RES)],
        out_shape=[jax.ShapeDtypeStruct((ng * nw * W,), jnp.float32)
                   for _ in range(RES)],
    )(t)


@functools.partial(jax.jit, static_argnums=(0, 1, 2, 3, 4))
def _biased_mf(B, K, N, NC, NS, user_ids, item_ids, bu, bv, U, V):
    NW = NC * NS
    bpw = B // NW
    nch = bpw // CH
    ngr = K // RES
    NP = -(-N // W) * W

    fu = _detile(U.T)
    fv = _detile(V.T)

    mesh = plsc.VectorSubcoreMesh(core_axis_name="c", subcore_axis_name="s")

    @functools.partial(
        pl.kernel,
        mesh=mesh,
        out_type=jax.ShapeDtypeStruct((B,), jnp.float32),
        compiler_params=pltpu.CompilerParams(needs_layout_passes=False,
                                             use_tc_tiling_on_sc=False),
        scratch_types=[
            pltpu.VMEM((nch, CH), jnp.int32),
            pltpu.VMEM((nch, CH), jnp.int32),
            pltpu.VMEM((ngr, nch, CH), jnp.int32),
            pltpu.VMEM((ngr, nch, CH), jnp.int32),
            pltpu.VMEM((K, nch, CH), jnp.float32),
            pltpu.VMEM((K, nch, CH), jnp.float32),
            pltpu.VMEM((nch, CH), jnp.float32),
            pltpu.VMEM((nch, CH), jnp.float32),
            pltpu.VMEM((bpw,), jnp.float32),
            pltpu.SemaphoreType.DMA,
        ],
    )
    def k(uids, iids, bu_t, bv_t, *rest):
        fu_t = rest[:RES]
        fv_t = rest[RES:2 * RES]
        out = rest[2 * RES]
        (uidx, iidx, ush, ish, ucol, vcol, bug, bvg, outv, sem) = \
            rest[2 * RES + 1:]
        wid = lax.axis_index("s") * NC + lax.axis_index("c")
        base = wid * bpw

        for c in range(nch):
            pltpu.sync_copy(uids.at[pl.ds(base + c * CH, CH)], uidx.at[c])
            pltpu.sync_copy(iids.at[pl.ds(base + c * CH, CH)], iidx.at[c])

        for g in range(ngr):
            for c in range(nch):
                for o in range(CH // L):
                    s = pl.ds(o * L, L)
                    ush[g, c, s] = uidx[c, s] + g * NP
                    ish[g, c, s] = iidx[c, s] + g * NP

        copies = []
        for c in range(nch):
            copies.append(pltpu.async_copy(
                bu_t.at[uidx.at[c]], bug.at[c], sem))
            copies.append(pltpu.async_copy(
                bv_t.at[iidx.at[c]], bvg.at[c], sem))
            for g in range(ngr):
                for r in range(RES):
                    j = g * RES + r
                    copies.append(pltpu.async_copy(
                        fu_t[r].at[ush.at[g, c]], ucol.at[j, c], sem))
                    copies.append(pltpu.async_copy(
                        fv_t[r].at[ish.at[g, c]], vcol.at[j, c], sem))
        for cp in copies:
            cp.wait()

        for c in range(nch):
            for o in range(CH // L):
                s = pl.ds(o * L, L)

                def body(j, acc):
                    return acc + ucol[j, c, s] * vcol[j, c, s]

                acc = lax.fori_loop(0, K, body, bug[c, s] + bvg[c, s] + GM)
                outv[pl.ds(c * CH + o * L, L)] = acc

        pltpu.sync_copy(outv, out.at[pl.ds(base, bpw)])

    return k(user_ids, item_ids, bu, bv, *fu, *fv)


def kernel(user_ids, item_ids, bu, bv, U, V):
    B = user_ids.shape[0]
    N, K = U.shape
    info = plsc.get_sparse_core_info()
    return _biased_mf(B, K, N, info.num_cores, info.num_subcores,
                      user_ids.astype(jnp.int32), item_ids.astype(jnp.int32),
                      bu.reshape(-1), bv.reshape(-1), U, V)
